# initial kernel scaffold (unmeasured)
import jax
import jax.numpy as jnp
from jax import lax
from jax.experimental import pallas as pl
from jax.experimental.pallas import tpu as pltpu


def kernel(
    x,
):
    def body(*refs):
        pass

    out_shape = jax.ShapeDtypeStruct(..., jnp.float32)
    return pl.pallas_call(body, out_shape=out_shape)(...)


# baseline (device time: 33887 ns/iter reference)
import jax
import jax.numpy as jnp
from jax import lax
from jax.experimental import pallas as pl
from jax.experimental.pallas import tpu as pltpu

N_DEV = 16


def kernel(x):
    m_per, n = x.shape

    def body(x_ref, out_ref, comm_ref, send_sems, recv_sems):
        my_pos = lax.axis_index("i")
        left = lax.rem(my_pos + N_DEV - 1, N_DEV)
        right = lax.rem(my_pos + 1, N_DEV)

        barrier_sem = pltpu.get_barrier_semaphore()
        for nbr in (left, right):
            pl.semaphore_signal(
                barrier_sem, inc=1,
                device_id=(nbr,), device_id_type=pl.DeviceIdType.MESH,
            )
        pl.semaphore_wait(barrier_sem, 2)

        xv = x_ref[:, :]
        val = jnp.max(xv, axis=0)
        rows = lax.broadcasted_iota(jnp.int32, (m_per, n), 0)
        idx_local = jnp.min(
            jnp.where(xv == val[None, :], rows, jnp.int32(m_per)), axis=0
        )
        gidx = (idx_local + my_pos * m_per).astype(jnp.float32)

        acc_val = val
        acc_idx = gidx
        comm_ref[0, 0, :] = val
        comm_ref[0, 1, :] = gidx

        for h in range(N_DEV - 1):
            send_slot = h % 2
            recv_slot = (h + 1) % 2
            rdma = pltpu.make_async_remote_copy(
                src_ref=comm_ref.at[send_slot],
                dst_ref=comm_ref.at[recv_slot],
                send_sem=send_sems.at[send_slot],
                recv_sem=recv_sems.at[recv_slot],
                device_id=(right,),
                device_id_type=pl.DeviceIdType.MESH,
            )
            rdma.start()
            rdma.wait()

            rv = comm_ref[recv_slot, 0, :]
            ri = comm_ref[recv_slot, 1, :]
            take = (rv > acc_val) | ((rv == acc_val) & (ri < acc_idx))
            acc_val = jnp.where(take, rv, acc_val)
            acc_idx = jnp.where(take, ri, acc_idx)

        out_ref[0, :] = acc_val
        out_ref[1, :] = acc_idx

    return pl.pallas_call(
        body,
        out_shape=jax.ShapeDtypeStruct((2, n), jnp.float32),
        in_specs=[pl.BlockSpec(memory_space=pltpu.VMEM)],
        out_specs=pl.BlockSpec(memory_space=pltpu.VMEM),
        scratch_shapes=[
            pltpu.VMEM((2, 2, n), jnp.float32),
            pltpu.SemaphoreType.DMA((2,)),
            pltpu.SemaphoreType.DMA((2,)),
        ],
        compiler_params=pltpu.CompilerParams(collective_id=0),
    )(x)


# device time: 10913 ns/iter; 3.1052x vs baseline; 3.1052x over previous
import jax
import jax.numpy as jnp
from jax import lax
from jax.experimental import pallas as pl
from jax.experimental.pallas import tpu as pltpu

N_DEV = 16


def kernel(x):
    m_per, n = x.shape

    def body(x_ref, out_ref, send_ref, recv_ref, send_sems, recv_sems):
        my_pos = lax.axis_index("i")

        barrier_sem = pltpu.get_barrier_semaphore()
        for j in range(N_DEV - 1):
            peer = lax.rem(my_pos + 1 + j, N_DEV)
            pl.semaphore_signal(
                barrier_sem, inc=1,
                device_id=(peer,), device_id_type=pl.DeviceIdType.MESH,
            )

        xv = x_ref[:, :]
        val = jnp.max(xv, axis=0)
        rows = lax.broadcasted_iota(jnp.int32, (m_per, n), 0)
        idx_local = jnp.min(
            jnp.where(xv == val[None, :], rows, jnp.int32(m_per)), axis=0
        )
        gidx = (idx_local + my_pos * m_per).astype(jnp.float32)
        send_ref[0, :] = val
        send_ref[1, :] = gidx

        pl.semaphore_wait(barrier_sem, N_DEV - 1)

        rdmas = []
        for j in range(N_DEV - 1):
            target = lax.rem(my_pos + 2 * N_DEV - 1 - j, N_DEV)
            rdma = pltpu.make_async_remote_copy(
                src_ref=send_ref,
                dst_ref=recv_ref.at[j],
                send_sem=send_sems.at[j],
                recv_sem=recv_sems.at[j],
                device_id=(target,),
                device_id_type=pl.DeviceIdType.MESH,
            )
            rdma.start()
            rdmas.append(rdma)

        acc_val = val
        acc_idx = gidx
        for j in range(N_DEV - 1):
            rdmas[j].wait_recv()
            rv = recv_ref[j, 0, :]
            ri = recv_ref[j, 1, :]
            take = (rv > acc_val) | ((rv == acc_val) & (ri < acc_idx))
            acc_val = jnp.where(take, rv, acc_val)
            acc_idx = jnp.where(take, ri, acc_idx)

        out_ref[0, :] = acc_val
        out_ref[1, :] = acc_idx

        for j in range(N_DEV - 1):
            rdmas[j].wait_send()

    return pl.pallas_call(
        body,
        out_shape=jax.ShapeDtypeStruct((2, n), jnp.float32),
        in_specs=[pl.BlockSpec(memory_space=pltpu.VMEM)],
        out_specs=pl.BlockSpec(memory_space=pltpu.VMEM),
        scratch_shapes=[
            pltpu.VMEM((2, n), jnp.float32),
            pltpu.VMEM((N_DEV - 1, 2, n), jnp.float32),
            pltpu.SemaphoreType.DMA((N_DEV - 1,)),
            pltpu.SemaphoreType.DMA((N_DEV - 1,)),
        ],
        compiler_params=pltpu.CompilerParams(collective_id=0),
    )(x)
